# Initial kernel scaffold; baseline (speedup 1.0000x reference)
import functools

import jax
import jax.numpy as jnp
from jax import lax
from jax.experimental import pallas as pl
from jax.experimental.pallas import tpu as pltpu
from jax.experimental.pallas import tpu_sc as plsc

_DIM = 16
_KL = 8
_CD = 128
_N = 50000
_V = 5000
_K = 6

_NC = 2
_NS = 16
_NW = _NC * _NS

_NP_CH = 112
_NP_NCH = 14
_NP = _NW * _NP_NCH * _NP_CH
_VP = 5120
_VROWS = _VP // _NS
_EP_CH = 120
_EP_NCH = 8
_EP = _NW * _EP_NCH * _EP_CH

_mesh = plsc.VectorSubcoreMesh(
    core_axis_name="c", subcore_axis_name="s", num_cores=_NC, num_subcores=_NS
)


def _sc_gather(table, idx3, *, nch, ch):
    d = table.shape[1]
    per_tile = nch * ch

    @functools.partial(
        pl.kernel,
        out_type=jax.ShapeDtypeStruct((_NW * per_tile, d), jnp.float32),
        mesh=_mesh,
        scratch_types=[
            pltpu.VMEM((ch,), jnp.int32),
            pltpu.VMEM((ch, d), jnp.float32),
            pltpu.SemaphoreType.DMA,
        ],
    )
    def body(table_hbm, idx_hbm, out_hbm, idx_v, rows_v, sem):
        cid = lax.axis_index("c")
        sid = lax.axis_index("s")
        wid = sid * _NC + cid
        base = wid * per_tile
        for c in range(nch):
            pltpu.sync_copy(idx_hbm.at[wid, c], idx_v)
            pltpu.async_copy(table_hbm.at[idx_v], rows_v, sem).wait()
            pltpu.sync_copy(rows_v, out_hbm.at[pl.ds(base + c * ch, ch)])

    return body(table, idx3)


def _sc_scatter_add(u_vals, ex_vals, idx3, zeros128, zeros16):
    @functools.partial(
        pl.kernel,
        out_type=(
            jax.ShapeDtypeStruct((_NC, _VP, _CD), jnp.float32),
            jax.ShapeDtypeStruct((_NC, _VP, _DIM), jnp.float32),
        ),
        mesh=_mesh,
        scratch_types=[
            pltpu.VMEM((_NP_CH,), jnp.int32),
            pltpu.VMEM((_NP_CH, _CD), jnp.float32),
            pltpu.VMEM((_NP_CH, _DIM), jnp.float32),
            pltpu.VMEM_SHARED((_VP, _CD), jnp.float32),
            pltpu.VMEM_SHARED((_VP, _DIM), jnp.float32),
            pltpu.SemaphoreType.DMA,
        ],
    )
    def body(u_hbm, ex_hbm, idx_hbm, z128_hbm, z16_hbm, out1_hbm, out2_hbm,
             idx_v, v1, v2, acc1_sh, acc2_sh, sem):
        cid = lax.axis_index("c")
        sid = lax.axis_index("s")
        wid = sid * _NC + cid
        row0 = sid * _VROWS
        pltpu.sync_copy(z128_hbm, acc1_sh.at[pl.ds(row0, _VROWS)])
        pltpu.sync_copy(z16_hbm, acc2_sh.at[pl.ds(row0, _VROWS)])
        plsc.subcore_barrier()
        base = wid * (_NP_NCH * _NP_CH)
        for c in range(_NP_NCH):
            pltpu.sync_copy(idx_hbm.at[wid, c], idx_v)
            pltpu.sync_copy(u_hbm.at[pl.ds(base + c * _NP_CH, _NP_CH)], v1)
            pltpu.sync_copy(ex_hbm.at[pl.ds(base + c * _NP_CH, _NP_CH)], v2)
            pltpu.sync_copy(v1, acc1_sh.at[idx_v], add=True)
            pltpu.sync_copy(v2, acc2_sh.at[idx_v], add=True)
        plsc.subcore_barrier()
        pltpu.sync_copy(acc1_sh.at[pl.ds(row0, _VROWS)],
                        out1_hbm.at[cid, pl.ds(row0, _VROWS)])
        pltpu.sync_copy(acc2_sh.at[pl.ds(row0, _VROWS)],
                        out2_hbm.at[cid, pl.ds(row0, _VROWS)])

    return body(u_vals, ex_vals, idx3, zeros128, zeros16)


def _mm(a, b):
    return jnp.dot(a, b, preferred_element_type=jnp.float32)


def _premlp_body(inp_ref, w0, b0, w1, b1, w2, b2, wsT, bs, sel816, e8x, t16,
                 x_ref, u_ref, ex_ref):
    x = inp_ref[...]
    x = jnp.maximum(_mm(x, w0[...]) + b0[...], 0.0)
    x = jnp.maximum(_mm(x, w1[...]) + b1[...], 0.0)
    x = _mm(x, w2[...]) + b2[...]
    score = _mm(x, wsT[...]) + bs[...]
    ex = jnp.exp(score)
    ex16 = _mm(ex, sel816[...])
    u = _mm(ex16, e8x[...]) * _mm(x, t16[...])
    x_ref[...] = x
    u_ref[...] = u
    ex_ref[...] = ex16


def _combine_body(p1_ref, p2_ref, e8x_ref, h_ref):
    u = p1_ref[0] + p1_ref[1]
    s2 = p2_ref[0] + p2_ref[1]
    denom = _mm(s2, e8x_ref[...])
    denom = jnp.where(denom == 0.0, 1.0, denom)
    h_ref[...] = jnp.maximum(u / denom, 0.0)


def _knn_body(pos_ref, posT_ref, out_ref, *, blk):
    i = pl.program_id(0)
    pb = pos_ref[...]
    pT = posT_ref[...]
    sqb = jnp.sum(pb * pb, axis=1, keepdims=True)
    sqa = jnp.sum(pT * pT, axis=0, keepdims=True)
    d = sqb + sqa - 2.0 * _mm(pb, pT)
    col = lax.broadcasted_iota(jnp.int32, (blk, _VP), 1)
    row = lax.broadcasted_iota(jnp.int32, (blk, _VP), 0) + i * blk
    d = jnp.where(col == row, d + 1e10, d)
    d = jnp.where(col >= _V, 3e38, d)
    res = jnp.zeros((blk, 8), jnp.int32)
    lane8 = lax.broadcasted_iota(jnp.int32, (blk, 8), 1)
    for t in range(_K):
        m = jnp.min(d, axis=1, keepdims=True)
        am = jnp.min(jnp.where(d == m, col, 2 ** 30), axis=1)
        res = jnp.where(lane8 == t, am[:, None], res)
        d = jnp.where(col == am[:, None], 3e38, d)
    out_ref[...] = res


def _delta_body(h_ref, pos_ref, wh1, bh1, s1, t1, wh2, bh2, s2, t2,
                w1p, b1f, out_ref):
    z1 = _mm(h_ref[...], wh1[...]) + bh1[...]
    y1 = jnp.maximum(z1, 0.0) * s1[...] + t1[...]
    z2 = _mm(y1, wh2[...]) + bh2[...]
    delta = jnp.maximum(z2, 0.0) * s2[...] + t2[...]
    out_ref[...] = _mm(delta - pos_ref[...], w1p[...]) + b1f[...]


def _edge_body(h_ref, dt_ref, xs_ref, ps_ref,
               w1x, s1f, t1f, w2f, b2f, s2f, t2f,
               wg1, bg1, sg1, tg1, wg2, bg2, sg2, tg2, w1p,
               out_ref, *, vblk):
    eblk = vblk * _K
    rep = jnp.broadcast_to(dt_ref[...][:, None, :], (vblk, _K, _CD))
    rep = rep.reshape(eblk, _CD)
    z1 = _mm(xs_ref[...], w1x[...]) + _mm(ps_ref[...], w1p[...]) + rep
    y1 = jnp.maximum(z1, 0.0) * s1f[...] + t1f[...]
    z2 = _mm(y1, w2f[...]) + b2f[...]
    msg = jnp.maximum(z2, 0.0) * s2f[...] + t2f[...]
    agg = jnp.max(msg.reshape(vblk, _K, _CD), axis=1)
    z3 = _mm(agg, wg1[...]) + bg1[...]
    y3 = jnp.maximum(z3, 0.0) * sg1[...] + tg1[...]
    z4 = _mm(y3, wg2[...]) + bg2[...]
    y4 = jnp.maximum(z4, 0.0) * sg2[...] + tg2[...]
    out_ref[...] = h_ref[...] + y4


def _mha_body(inp_ref, x_ref, hs_ref, sn, tn, wqT, bq, bdk, bk, bdv, bv,
              t16, g8, e8, hmat, woT, bo, out_ref):
    hs = hs_ref[...] * sn[...] + tn[...]
    q = _mm(x_ref[...], wqT[...]) + bq[...]
    kf = _mm(hs, bdk[...]) + bk[...]
    vf = _mm(hs, bdv[...]) + bv[...]
    qrep = _mm(q, t16[...])
    lg = _mm(qrep * kf, g8[...]) * 0.25
    mx = jnp.max(lg, axis=1, keepdims=True)
    e = jnp.exp(lg - mx)
    a = e / jnp.sum(e, axis=1, keepdims=True)
    o16 = _mm(_mm(a, e8[...]) * vf, hmat[...])
    o = _mm(o16, woT[...]) + bo[...]
    out_ref[...] = jnp.concatenate([inp_ref[...], o], axis=1)


def _bn_fold(p, eps):
    s = p["gamma"] / jnp.sqrt(p["var"] + eps)
    t = p["beta"] - p["mean"] * s
    return s, t


def _row(v):
    return v.reshape(1, -1)


def _mymlp_fold(layers):
    out = []
    for l in layers:
        s, t = _bn_fold(l["bn"], 1e-5)
        out.append((l["lin"]["W"].T, _row(l["lin"]["b"]), _row(s), _row(t)))
    return out


def _full_spec(shape):
    nd = len(shape)
    return pl.BlockSpec(shape, lambda i, _n=nd: (0,) * _n)


def _blk_spec(shape):
    return pl.BlockSpec(shape, lambda i: (i, 0))


def _call(body, grid, in_arrays, in_specs, out_shapes, out_specs, **kw):
    return pl.pallas_call(
        functools.partial(body, **kw) if kw else body,
        grid=grid,
        in_specs=in_specs,
        out_specs=out_specs,
        out_shape=out_shapes,
    )(*in_arrays)


def kernel(inp, inverse, coords, bev_shape, params):
    f32 = jnp.float32
    eye8 = jnp.eye(_KL, dtype=f32)
    eye16 = jnp.eye(_DIM, dtype=f32)
    e8 = jnp.kron(eye8, jnp.ones((1, _DIM), f32))
    e8x = jnp.concatenate([e8, jnp.zeros((8, _CD), f32)], axis=0)
    hmat = jnp.kron(jnp.ones((_KL, 1), f32), eye16)
    t16 = hmat.T
    g8 = e8.T
    sel816 = jnp.concatenate([eye8, jnp.zeros((_KL, _KL), f32)], axis=1)

    pre = []
    for l in params["pre_mlp"]:
        s, t = _bn_fold(l["bn"], 1e-3)
        pre.append((l["lin"]["W"].T * s[None, :], _row(l["lin"]["b"] * s + t)))
    wsT = params["score"]["W"].T
    bs = _row(params["score"]["b"])

    nblk, blk = 25, 2000
    consts_a = (pre[0][0], pre[0][1], pre[1][0], pre[1][1], pre[2][0],
                pre[2][1], wsT, bs, sel816, e8x, t16)
    x, u_vals, ex_vals = _call(
        _premlp_body, (nblk,),
        [inp, *consts_a],
        [_blk_spec((blk, _DIM))] + [_full_spec(a.shape) for a in consts_a],
        (jax.ShapeDtypeStruct((_N, _DIM), f32),
         jax.ShapeDtypeStruct((_N, _CD), f32),
         jax.ShapeDtypeStruct((_N, _DIM), f32)),
        (_blk_spec((blk, _DIM)), _blk_spec((blk, _CD)), _blk_spec((blk, _DIM))),
    )

    pad_n = _NP - _N
    u_pad = jnp.concatenate([u_vals, jnp.zeros((pad_n, _CD), f32)], axis=0)
    ex_pad = jnp.concatenate([ex_vals, jnp.zeros((pad_n, _DIM), f32)], axis=0)
    inv_pad = jnp.concatenate(
        [inverse.astype(jnp.int32), jnp.zeros((pad_n,), jnp.int32)])
    inv3 = inv_pad.reshape(_NW, _NP_NCH, _NP_CH)
    z128 = jnp.zeros((_VROWS, _CD), f32)
    z16 = jnp.zeros((_VROWS, _DIM), f32)
    p1, p2 = _sc_scatter_add(u_pad, ex_pad, inv3, z128, z16)

    h = _call(
        _combine_body, (1,),
        [p1, p2, e8x],
        [_full_spec(p1.shape), _full_spec(p2.shape), _full_spec(e8x.shape)],
        jax.ShapeDtypeStruct((_VP, _CD), f32),
        _full_spec((_VP, _CD)),
    )

    pos = coords[:, 1:4]
    pos16 = jnp.zeros((_VP, _DIM), f32).at[:_V, :3].set(pos)
    posT = pos16.T
    kblk = 500
    knn = _call(
        _knn_body, (_V // kblk,),
        [pos16[:_V], posT],
        [_blk_spec((kblk, _DIM)), _full_spec(posT.shape)],
        jax.ShapeDtypeStruct((_V, 8), jnp.int32),
        _blk_spec((kblk, 8)),
        blk=kblk,
    )
    src = knn[:, :_K].reshape(-1)
    src3 = jnp.concatenate(
        [src, jnp.zeros((_EP - src.shape[0],), jnp.int32)]
    ).reshape(_NW, _EP_NCH, _EP_CH)

    pos_src = _sc_gather(pos16, src3, nch=_EP_NCH, ch=_EP_CH)

    vblk = 256
    ngrid = _VP // vblk
    for lp in params["gnn"]:
        hf = _mymlp_fold(lp["h"])
        ff = _mymlp_fold(lp["f"])
        gf = _mymlp_fold(lp["g"])
        (wh1, bh1, s1, t1), (wh2r, bh2r, s2r, t2r) = hf
        wh2 = jnp.zeros((64, _DIM), f32).at[:, :3].set(wh2r)
        bh2 = jnp.zeros((1, _DIM), f32).at[:, :3].set(bh2r)
        s2 = jnp.zeros((1, _DIM), f32).at[:, :3].set(s2r)
        t2 = jnp.zeros((1, _DIM), f32).at[:, :3].set(t2r)
        (w1T, b1, s1f, t1f), (w2f, b2f, s2f, t2f) = ff
        w1p = jnp.zeros((_DIM, _CD), f32).at[:3, :].set(w1T[:3, :])
        w1x = w1T[3:, :]
        (wg1, bg1, sg1, tg1), (wg2, bg2, sg2, tg2) = gf

        consts_d = (wh1, bh1, s1, t1, wh2, bh2, s2, t2, w1p, b1)
        dstterm = _call(
            _delta_body, (1,),
            [h, pos16, *consts_d],
            [_full_spec((_VP, _CD)), _full_spec((_VP, _DIM))] +
            [_full_spec(a.shape) for a in consts_d],
            jax.ShapeDtypeStruct((_VP, _CD), f32),
            _full_spec((_VP, _CD)),
        )
        x_src = _sc_gather(h, src3, nch=_EP_NCH, ch=_EP_CH)
        consts_e = (w1x, s1f, t1f, w2f, b2f, s2f, t2f,
                    wg1, bg1, sg1, tg1, wg2, bg2, sg2, tg2, w1p)
        h = _call(
            _edge_body, (ngrid,),
            [h, dstterm, x_src, pos_src, *consts_e],
            [_blk_spec((vblk, _CD)), _blk_spec((vblk, _CD)),
             _blk_spec((vblk * _K, _CD)), _blk_spec((vblk * _K, _DIM))] +
            [_full_spec(a.shape) for a in consts_e],
            jax.ShapeDtypeStruct((_VP, _CD), f32),
            _blk_spec((vblk, _CD)),
            vblk=vblk,
        )

    hs = _sc_gather(h, inv3, nch=_NP_NCH, ch=_NP_CH)[:_N]

    sn, tn = _bn_fold(params["norm"], 1e-3)
    sn128 = _row(jnp.tile(sn, _KL))
    tn128 = _row(jnp.tile(tn, _KL))
    m = params["mha"]
    wq, wk, wv = jnp.split(m["in_W"], 3, axis=0)
    bq, bk, bv = jnp.split(m["in_b"], 3)
    bdk = jnp.kron(eye8, wk.T)
    bdv = jnp.kron(eye8, wv.T)
    bk128 = _row(jnp.tile(bk, _KL))
    bv128 = _row(jnp.tile(bv, _KL))
    consts_g = (sn128, tn128, wq.T, _row(bq), bdk, bk128, bdv, bv128,
                t16, g8, e8, hmat, m["out_W"].T, _row(m["out_b"]))
    out = _call(
        _mha_body, (nblk,),
        [inp, x, hs, *consts_g],
        [_blk_spec((blk, _DIM)), _blk_spec((blk, _DIM)),
         _blk_spec((blk, _CD))] +
        [_full_spec(a.shape) for a in consts_g],
        jax.ShapeDtypeStruct((_N, 2 * _DIM), f32),
        _blk_spec((blk, 2 * _DIM)),
    )
    return out


# speedup vs baseline: 4.9488x; 4.9488x over previous
import functools

import jax
import jax.numpy as jnp
from jax import lax
from jax.experimental import pallas as pl
from jax.experimental.pallas import tpu as pltpu
from jax.experimental.pallas import tpu_sc as plsc

_DIM = 16
_KL = 8
_CD = 128
_N = 50000
_V = 5000
_K = 6

_NC = 2
_NS = 16
_NW = _NC * _NS

_NP_CH = 112
_NP_NCH = 14
_NP = _NW * _NP_NCH * _NP_CH
_VP = 5120
_VROWS = _VP // _NS
_EP_CH = 120
_EP_NCH = 8
_EP = _NW * _EP_NCH * _EP_CH

@functools.cache
def _mesh():
    return plsc.VectorSubcoreMesh(
        core_axis_name="c", subcore_axis_name="s",
        num_cores=_NC, num_subcores=_NS,
    )


def _sc_gather(table, idx3, *, nch, ch):
    d = table.shape[1]
    per_tile = nch * ch

    @functools.partial(
        pl.kernel,
        out_type=jax.ShapeDtypeStruct((_NW * per_tile, d), jnp.float32),
        mesh=_mesh(),
        scratch_types=[
            pltpu.VMEM((ch,), jnp.int32),
            pltpu.VMEM((ch, d), jnp.float32),
            pltpu.SemaphoreType.DMA,
        ],
    )
    def body(table_hbm, idx_hbm, out_hbm, idx_v, rows_v, sem):
        cid = lax.axis_index("c")
        sid = lax.axis_index("s")
        wid = sid * _NC + cid
        base = wid * per_tile
        for c in range(nch):
            pltpu.sync_copy(idx_hbm.at[wid, c], idx_v)
            pltpu.async_copy(table_hbm.at[idx_v], rows_v, sem).wait()
            pltpu.sync_copy(rows_v, out_hbm.at[pl.ds(base + c * ch, ch)])

    return body(table, idx3)


def _sc_scatter_add(u_vals, ex_vals, idx3, zeros128):
    @functools.partial(
        pl.kernel,
        out_type=(
            jax.ShapeDtypeStruct((_NC, _VP, _CD), jnp.float32),
            jax.ShapeDtypeStruct((_NC, _VP, _CD), jnp.float32),
        ),
        mesh=_mesh(),
        scratch_types=[
            pltpu.VMEM((_NP_CH,), jnp.int32),
            pltpu.VMEM((_NP_CH, _CD), jnp.float32),
            pltpu.VMEM((_NP_CH, _CD), jnp.float32),
            pltpu.VMEM_SHARED((_VP, _CD), jnp.float32),
            pltpu.VMEM_SHARED((_VP, _CD), jnp.float32),
            pltpu.SemaphoreType.DMA,
        ],
    )
    def body(u_hbm, ex_hbm, idx_hbm, z_hbm, out1_hbm, out2_hbm,
             idx_v, v1, v2, acc1_sh, acc2_sh, sem):
        cid = lax.axis_index("c")
        sid = lax.axis_index("s")
        wid = sid * _NC + cid
        row0 = sid * _VROWS
        pltpu.sync_copy(z_hbm, acc1_sh.at[pl.ds(row0, _VROWS)])
        pltpu.sync_copy(z_hbm, acc2_sh.at[pl.ds(row0, _VROWS)])
        plsc.subcore_barrier()
        base = wid * (_NP_NCH * _NP_CH)
        for c in range(_NP_NCH):
            pltpu.sync_copy(idx_hbm.at[wid, c], idx_v)
            pltpu.sync_copy(u_hbm.at[pl.ds(base + c * _NP_CH, _NP_CH)], v1)
            pltpu.sync_copy(ex_hbm.at[pl.ds(base + c * _NP_CH, _NP_CH)], v2)
            pltpu.sync_copy(v1, acc1_sh.at[idx_v], add=True)
            pltpu.sync_copy(v2, acc2_sh.at[idx_v], add=True)
        plsc.subcore_barrier()
        pltpu.sync_copy(acc1_sh.at[pl.ds(row0, _VROWS)],
                        out1_hbm.at[cid, pl.ds(row0, _VROWS)])
        pltpu.sync_copy(acc2_sh.at[pl.ds(row0, _VROWS)],
                        out2_hbm.at[cid, pl.ds(row0, _VROWS)])

    return body(u_vals, ex_vals, idx3, zeros128)


def _mm(a, b):
    return jnp.dot(a, b, preferred_element_type=jnp.float32)


def _premlp_body(inp_ref, w0, b0, w1, b1, w2, b2, wsT, bs, sel816, e8x, t16,
                 widen, x_ref, u_ref, ex_ref):
    x = inp_ref[...]
    x = jnp.maximum(_mm(x, w0[...]) + b0[...], 0.0)
    x = jnp.maximum(_mm(x, w1[...]) + b1[...], 0.0)
    x = _mm(x, w2[...]) + b2[...]
    score = _mm(x, wsT[...]) + bs[...]
    ex = jnp.exp(score)
    ex16 = _mm(ex, sel816[...])
    u = _mm(ex16, e8x[...]) * _mm(x, t16[...])
    x_ref[...] = x
    u_ref[...] = u
    ex_ref[...] = _mm(ex16, widen[...])


def _combine_body(p1_ref, p2_ref, selex_ref, h_ref):
    u = p1_ref[0] + p1_ref[1]
    s2 = p2_ref[0] + p2_ref[1]
    denom = _mm(s2, selex_ref[...])
    denom = jnp.where(denom == 0.0, 1.0, denom)
    h_ref[...] = jnp.maximum(u / denom, 0.0)


def _knn_body(pos_ref, posT_ref, out_ref, *, blk):
    i = pl.program_id(0)
    pb = pos_ref[...]
    pT = posT_ref[...]
    sqb = jnp.sum(pb * pb, axis=1, keepdims=True)
    sqa = jnp.sum(pT * pT, axis=0, keepdims=True)
    d = sqb + sqa - 2.0 * _mm(pb, pT)
    col = lax.broadcasted_iota(jnp.int32, (blk, _VP), 1)
    row = lax.broadcasted_iota(jnp.int32, (blk, _VP), 0) + i * blk
    d = jnp.where(col == row, d + 1e10, d)
    d = jnp.where(col >= _V, 3e38, d)
    res = jnp.zeros((blk, 8), jnp.int32)
    lane8 = lax.broadcasted_iota(jnp.int32, (blk, 8), 1)
    for t in range(_K):
        m = jnp.min(d, axis=1, keepdims=True)
        am = jnp.min(jnp.where(d == m, col, 2 ** 30), axis=1)
        res = jnp.where(lane8 == t, am[:, None], res)
        d = jnp.where(col == am[:, None], 3e38, d)
    out_ref[...] = res


def _delta_body(h_ref, pos_ref, wh1, bh1, s1, t1, wh2, bh2, s2, t2,
                w1p, b1f, out_ref):
    z1 = _mm(h_ref[...], wh1[...]) + bh1[...]
    y1 = jnp.maximum(z1, 0.0) * s1[...] + t1[...]
    z2 = _mm(y1, wh2[...]) + bh2[...]
    delta = jnp.maximum(z2, 0.0) * s2[...] + t2[...]
    out_ref[...] = _mm(delta - pos_ref[...], w1p[...]) + b1f[...]


def _edge_body(h_ref, dt_ref, xs_ref, ps_ref,
               w1x, s1f, t1f, w2f, b2f, s2f, t2f,
               wg1, bg1, sg1, tg1, wg2, bg2, sg2, tg2, w1p,
               out_ref, *, vblk):
    eblk = vblk * _K
    rep = jnp.broadcast_to(dt_ref[...][:, None, :], (vblk, _K, _CD))
    rep = rep.reshape(eblk, _CD)
    z1 = _mm(xs_ref[...], w1x[...]) + _mm(ps_ref[...], w1p[...]) + rep
    y1 = jnp.maximum(z1, 0.0) * s1f[...] + t1f[...]
    z2 = _mm(y1, w2f[...]) + b2f[...]
    msg = jnp.maximum(z2, 0.0) * s2f[...] + t2f[...]
    agg = jnp.max(msg.reshape(vblk, _K, _CD), axis=1)
    z3 = _mm(agg, wg1[...]) + bg1[...]
    y3 = jnp.maximum(z3, 0.0) * sg1[...] + tg1[...]
    z4 = _mm(y3, wg2[...]) + bg2[...]
    y4 = jnp.maximum(z4, 0.0) * sg2[...] + tg2[...]
    out_ref[...] = h_ref[...] + y4


def _mha_body(inp_ref, x_ref, hs_ref, sn, tn, wqT, bq, bdk, bk, bdv, bv,
              t16, g8, e8, hmat, woT, bo, out_ref):
    hs = hs_ref[...] * sn[...] + tn[...]
    q = _mm(x_ref[...], wqT[...]) + bq[...]
    kf = _mm(hs, bdk[...]) + bk[...]
    vf = _mm(hs, bdv[...]) + bv[...]
    qrep = _mm(q, t16[...])
    lg = _mm(qrep * kf, g8[...]) * 0.25
    mx = jnp.max(lg, axis=1, keepdims=True)
    e = jnp.exp(lg - mx)
    a = e / jnp.sum(e, axis=1, keepdims=True)
    o16 = _mm(_mm(a, e8[...]) * vf, hmat[...])
    o = _mm(o16, woT[...]) + bo[...]
    out_ref[...] = jnp.concatenate([inp_ref[...], o], axis=1)


def _bn_fold(p, eps):
    s = p["gamma"] / jnp.sqrt(p["var"] + eps)
    t = p["beta"] - p["mean"] * s
    return s, t


def _row(v):
    return v.reshape(1, -1)


def _mymlp_fold(layers):
    out = []
    for l in layers:
        s, t = _bn_fold(l["bn"], 1e-5)
        out.append((l["lin"]["W"].T, _row(l["lin"]["b"]), _row(s), _row(t)))
    return out


def _full_spec(shape):
    nd = len(shape)
    return pl.BlockSpec(shape, lambda i, _n=nd: (0,) * _n)


def _blk_spec(shape):
    return pl.BlockSpec(shape, lambda i: (i, 0))


def _call(body, grid, in_arrays, in_specs, out_shapes, out_specs, **kw):
    return pl.pallas_call(
        functools.partial(body, **kw) if kw else body,
        grid=grid,
        in_specs=in_specs,
        out_specs=out_specs,
        out_shape=out_shapes,
    )(*in_arrays)


def kernel(inp, inverse, coords, bev_shape, params):
    f32 = jnp.float32
    eye8 = jnp.eye(_KL, dtype=f32)
    eye16 = jnp.eye(_DIM, dtype=f32)
    e8 = jnp.kron(eye8, jnp.ones((1, _DIM), f32))
    e8x = jnp.concatenate([e8, jnp.zeros((8, _CD), f32)], axis=0)
    hmat = jnp.kron(jnp.ones((_KL, 1), f32), eye16)
    t16 = hmat.T
    g8 = e8.T
    sel816 = jnp.concatenate([eye8, jnp.zeros((_KL, _KL), f32)], axis=1)

    pre = []
    for l in params["pre_mlp"]:
        s, t = _bn_fold(l["bn"], 1e-3)
        pre.append((l["lin"]["W"].T * s[None, :], _row(l["lin"]["b"] * s + t)))
    wsT = params["score"]["W"].T
    bs = _row(params["score"]["b"])

    nblk, blk = 25, 2000
    widen = jnp.eye(_DIM, _CD, dtype=f32)
    consts_a = (pre[0][0], pre[0][1], pre[1][0], pre[1][1], pre[2][0],
                pre[2][1], wsT, bs, sel816, e8x, t16, widen)
    x, u_vals, ex_vals = _call(
        _premlp_body, (nblk,),
        [inp, *consts_a],
        [_blk_spec((blk, _DIM))] + [_full_spec(a.shape) for a in consts_a],
        (jax.ShapeDtypeStruct((_N, _DIM), f32),
         jax.ShapeDtypeStruct((_N, _CD), f32),
         jax.ShapeDtypeStruct((_N, _CD), f32)),
        (_blk_spec((blk, _DIM)), _blk_spec((blk, _CD)), _blk_spec((blk, _CD))),
    )

    pad_n = _NP - _N
    u_pad = jnp.concatenate([u_vals, jnp.zeros((pad_n, _CD), f32)], axis=0)
    ex_pad = jnp.concatenate([ex_vals, jnp.zeros((pad_n, _CD), f32)], axis=0)
    inv_pad = jnp.concatenate(
        [inverse.astype(jnp.int32), jnp.zeros((pad_n,), jnp.int32)])
    inv3 = inv_pad.reshape(_NW, _NP_NCH, _NP_CH)
    z128 = jnp.zeros((_VROWS, _CD), f32)
    p1, p2 = _sc_scatter_add(u_pad, ex_pad, inv3, z128)

    selex = jnp.concatenate([e8x, jnp.zeros((_CD - _DIM, _CD), f32)], axis=0)
    h = _call(
        _combine_body, (1,),
        [p1, p2, selex],
        [_full_spec(p1.shape), _full_spec(p2.shape), _full_spec(selex.shape)],
        jax.ShapeDtypeStruct((_VP, _CD), f32),
        _full_spec((_VP, _CD)),
    )

    pos = coords[:, 1:4]
    pos16 = jnp.zeros((_VP, _DIM), f32).at[:_V, :3].set(pos)
    posT = pos16.T
    kblk = 1000
    knn = _call(
        _knn_body, (_V // kblk,),
        [pos16[:_V], posT],
        [_blk_spec((kblk, _DIM)), _full_spec(posT.shape)],
        jax.ShapeDtypeStruct((_V, 8), jnp.int32),
        _blk_spec((kblk, 8)),
        blk=kblk,
    )
    src = knn[:, :_K].reshape(-1)
    src3 = jnp.concatenate(
        [src, jnp.zeros((_EP - src.shape[0],), jnp.int32)]
    ).reshape(_NW, _EP_NCH, _EP_CH)

    pos128 = jnp.zeros((_VP, _CD), f32).at[:_V, :3].set(pos)
    pos_src = _sc_gather(pos128, src3, nch=_EP_NCH, ch=_EP_CH)

    vblk = 256
    ngrid = _VP // vblk
    for lp in params["gnn"]:
        hf = _mymlp_fold(lp["h"])
        ff = _mymlp_fold(lp["f"])
        gf = _mymlp_fold(lp["g"])
        (wh1, bh1, s1, t1), (wh2r, bh2r, s2r, t2r) = hf
        wh2 = jnp.zeros((64, _DIM), f32).at[:, :3].set(wh2r)
        bh2 = jnp.zeros((1, _DIM), f32).at[:, :3].set(bh2r)
        s2 = jnp.zeros((1, _DIM), f32).at[:, :3].set(s2r)
        t2 = jnp.zeros((1, _DIM), f32).at[:, :3].set(t2r)
        (w1T, b1, s1f, t1f), (w2f, b2f, s2f, t2f) = ff
        w1p = jnp.zeros((_DIM, _CD), f32).at[:3, :].set(w1T[:3, :])
        w1pe = jnp.zeros((_CD, _CD), f32).at[:3, :].set(w1T[:3, :])
        w1x = w1T[3:, :]
        (wg1, bg1, sg1, tg1), (wg2, bg2, sg2, tg2) = gf

        consts_d = (wh1, bh1, s1, t1, wh2, bh2, s2, t2, w1p, b1)
        dstterm = _call(
            _delta_body, (1,),
            [h, pos16, *consts_d],
            [_full_spec((_VP, _CD)), _full_spec((_VP, _DIM))] +
            [_full_spec(a.shape) for a in consts_d],
            jax.ShapeDtypeStruct((_VP, _CD), f32),
            _full_spec((_VP, _CD)),
        )
        x_src = _sc_gather(h, src3, nch=_EP_NCH, ch=_EP_CH)
        consts_e = (w1x, s1f, t1f, w2f, b2f, s2f, t2f,
                    wg1, bg1, sg1, tg1, wg2, bg2, sg2, tg2, w1pe)
        h = _call(
            _edge_body, (ngrid,),
            [h, dstterm, x_src, pos_src, *consts_e],
            [_blk_spec((vblk, _CD)), _blk_spec((vblk, _CD)),
             _blk_spec((vblk * _K, _CD)), _blk_spec((vblk * _K, _CD))] +
            [_full_spec(a.shape) for a in consts_e],
            jax.ShapeDtypeStruct((_VP, _CD), f32),
            _blk_spec((vblk, _CD)),
            vblk=vblk,
        )

    hs = _sc_gather(h, inv3, nch=_NP_NCH, ch=_NP_CH)[:_N]

    sn, tn = _bn_fold(params["norm"], 1e-3)
    sn128 = _row(jnp.tile(sn, _KL))
    tn128 = _row(jnp.tile(tn, _KL))
    m = params["mha"]
    wq, wk, wv = jnp.split(m["in_W"], 3, axis=0)
    bq, bk, bv = jnp.split(m["in_b"], 3)
    bdk = jnp.kron(eye8, wk.T)
    bdv = jnp.kron(eye8, wv.T)
    bk128 = _row(jnp.tile(bk, _KL))
    bv128 = _row(jnp.tile(bv, _KL))
    consts_g = (sn128, tn128, wq.T, _row(bq), bdk, bk128, bdv, bv128,
                t16, g8, e8, hmat, m["out_W"].T, _row(m["out_b"]))
    out = _call(
        _mha_body, (nblk,),
        [inp, x, hs, *consts_g],
        [_blk_spec((blk, _DIM)), _blk_spec((blk, _DIM)),
         _blk_spec((blk, _CD))] +
        [_full_spec(a.shape) for a in consts_g],
        jax.ShapeDtypeStruct((_N, 2 * _DIM), f32),
        _blk_spec((blk, 2 * _DIM)),
    )
    return out


# speedup vs baseline: 7.6278x; 1.5413x over previous
import functools

import jax
import jax.numpy as jnp
from jax import lax
from jax.experimental import pallas as pl
from jax.experimental.pallas import tpu as pltpu
from jax.experimental.pallas import tpu_sc as plsc

_DIM = 16
_KL = 8
_CD = 128
_N = 50000
_V = 5000
_K = 6

_NC = 2
_NS = 16
_NW = _NC * _NS

_SC_CH = 224
_SC_N = 14
_NP_CH = 224
_NP_NCH = 7
_NP = _NW * _NP_NCH * _NP_CH
_VP = 5120
_VROWS = _VP // _NS
_EP_CH = 192
_EP_NCH = 5
_EP = _NW * _EP_NCH * _EP_CH

@functools.cache
def _mesh():
    return plsc.VectorSubcoreMesh(
        core_axis_name="c", subcore_axis_name="s",
        num_cores=_NC, num_subcores=_NS,
    )


def _sc_gather(table, idx2, *, nch, ch):
    d = table.shape[1]
    per_tile = nch * ch

    @functools.partial(
        pl.kernel,
        out_type=jax.ShapeDtypeStruct((_NW * per_tile, d), jnp.float32),
        mesh=_mesh(),
        scratch_types=[
            pltpu.VMEM((per_tile,), jnp.int32),
            pltpu.VMEM((ch, d), jnp.float32),
            pltpu.VMEM((ch, d), jnp.float32),
            pltpu.VMEM_SHARED((_VP, d), jnp.float32),
            pltpu.SemaphoreType.DMA,
            pltpu.SemaphoreType.DMA,
        ],
    )
    def body(table_hbm, idx_hbm, out_hbm, idx_full, rows0, rows1, tab_sh,
             sem0, sem1):
        cid = lax.axis_index("c")
        sid = lax.axis_index("s")
        wid = sid * _NC + cid
        base = wid * per_tile
        pltpu.sync_copy(table_hbm.at[pl.ds(sid * _VROWS, _VROWS)],
                        tab_sh.at[pl.ds(sid * _VROWS, _VROWS)])
        pltpu.sync_copy(idx_hbm.at[wid], idx_full)
        plsc.subcore_barrier()
        rows = (rows0, rows1)
        sems = (sem0, sem1)
        cps = [None] * nch
        cps[0] = pltpu.async_copy(
            tab_sh.at[idx_full.at[pl.ds(0, ch)]], rows0, sem0)
        for c in range(nch):
            b = c & 1
            nb = 1 - b
            if c + 1 < nch:
                cps[c + 1] = pltpu.async_copy(
                    tab_sh.at[idx_full.at[pl.ds((c + 1) * ch, ch)]],
                    rows[nb], sems[nb])
            cps[c].wait()
            pltpu.sync_copy(rows[b], out_hbm.at[pl.ds(base + c * ch, ch)])

    return body(table, idx2)


def _sc_scatter_add(u_vals, ex_vals, idx2, zeros128):
    @functools.partial(
        pl.kernel,
        out_type=(
            jax.ShapeDtypeStruct((_VP, _CD), jnp.float32),
            jax.ShapeDtypeStruct((_VP, _CD), jnp.float32),
        ),
        mesh=_mesh(),
        scratch_types=[
            pltpu.VMEM((_SC_N * _SC_CH,), jnp.int32),
            pltpu.VMEM((_SC_CH, _CD), jnp.float32),
            pltpu.VMEM((_SC_CH, _CD), jnp.float32),
            pltpu.VMEM_SHARED((_VP, _CD), jnp.float32),
            pltpu.SemaphoreType.DMA,
            pltpu.SemaphoreType.DMA,
        ],
    )
    def body(u_hbm, ex_hbm, idx_hbm, z_hbm, out1_hbm, out2_hbm,
             idx_full, v0, v1, acc_sh, sem0, sem1):
        cid = lax.axis_index("c")
        sid = lax.axis_index("s")
        row0 = sid * _VROWS
        pltpu.sync_copy(z_hbm, acc_sh.at[pl.ds(row0, _VROWS)])
        plsc.subcore_barrier()
        pltpu.sync_copy(idx_hbm.at[sid], idx_full)
        base = sid * (_SC_N * _SC_CH)
        bufs = (v0, v1)
        sems = (sem0, sem1)

        def run(vals_hbm, out_hbm):
            cps = [None] * _SC_N
            cps[0] = pltpu.async_copy(
                vals_hbm.at[pl.ds(base, _SC_CH)], v0, sem0)
            for c in range(_SC_N):
                b = c & 1
                nb = 1 - b
                if c + 1 < _SC_N:
                    cps[c + 1] = pltpu.async_copy(
                        vals_hbm.at[pl.ds(base + (c + 1) * _SC_CH, _SC_CH)],
                        bufs[nb], sems[nb])
                cps[c].wait()
                pltpu.sync_copy(
                    bufs[b],
                    acc_sh.at[idx_full.at[pl.ds(c * _SC_CH, _SC_CH)]],
                    add=True)
            plsc.subcore_barrier()
            pltpu.sync_copy(acc_sh.at[pl.ds(row0, _VROWS)],
                            out_hbm.at[pl.ds(row0, _VROWS)])

        @pl.when(cid == 0)
        def _():
            run(u_hbm, out1_hbm)

        @pl.when(cid == 1)
        def _():
            run(ex_hbm, out2_hbm)

    return body(u_vals, ex_vals, idx2, zeros128)


def _mm(a, b):
    return jnp.dot(a, b, preferred_element_type=jnp.float32)


def _premlp_body(inp_ref, w0, b0, w1, b1, w2, b2, wsT, bs, sel816, e8x, t16,
                 widen, x_ref, u_ref, ex_ref, *, blk):
    pid = pl.program_id(0)
    x = inp_ref[...]
    x = jnp.maximum(_mm(x, w0[...]) + b0[...], 0.0)
    x = jnp.maximum(_mm(x, w1[...]) + b1[...], 0.0)
    x = _mm(x, w2[...]) + b2[...]
    score = _mm(x, wsT[...]) + bs[...]
    valid = (lax.broadcasted_iota(jnp.int32, (blk, 1), 0) + pid * blk) < _N
    ex = jnp.where(valid, jnp.exp(score), 0.0)
    ex16 = _mm(ex, sel816[...])
    u = _mm(ex16, e8x[...]) * _mm(x, t16[...])
    x_ref[...] = x
    u_ref[...] = jnp.where(valid, u, 0.0)
    ex_ref[...] = _mm(ex16, widen[...])


def _combine_body(p1_ref, p2_ref, selex_ref, h_ref):
    u = p1_ref[...]
    s2 = p2_ref[...]
    denom = _mm(s2, selex_ref[...])
    denom = jnp.where(denom == 0.0, 1.0, denom)
    h_ref[...] = jnp.maximum(u / denom, 0.0)


def _knn_body(pos_ref, posT_ref, out_ref, *, blk):
    i = pl.program_id(0)
    pb = pos_ref[...]
    pT = posT_ref[...]
    sqb = jnp.sum(pb * pb, axis=1, keepdims=True)
    sqa = jnp.sum(pT * pT, axis=0, keepdims=True)
    d = sqb + sqa - 2.0 * _mm(pb, pT)
    col = lax.broadcasted_iota(jnp.int32, (blk, _VP), 1)
    row = lax.broadcasted_iota(jnp.int32, (blk, _VP), 0) + i * blk
    d = jnp.where(col == row, d + 1e10, d)
    d = jnp.where(col >= _V, 3e38, d)
    res = jnp.zeros((blk, 8), jnp.int32)
    lane8 = lax.broadcasted_iota(jnp.int32, (blk, 8), 1)
    for t in range(_K):
        am = jnp.argmin(d, axis=1).astype(jnp.int32)
        res = jnp.where(lane8 == t, am[:, None], res)
        d = jnp.where(col == am[:, None], 3e38, d)
    out_ref[...] = res


def _delta_body(h_ref, pos_ref, wh1, bh1, s1, t1, wh2, bh2, s2, t2,
                w1p, b1f, out_ref):
    z1 = _mm(h_ref[...], wh1[...]) + bh1[...]
    y1 = jnp.maximum(z1, 0.0) * s1[...] + t1[...]
    z2 = _mm(y1, wh2[...]) + bh2[...]
    delta = jnp.maximum(z2, 0.0) * s2[...] + t2[...]
    out_ref[...] = _mm(delta - pos_ref[...], w1p[...]) + b1f[...]


def _edge_body(h_ref, dt_ref, xs_ref, ps_ref,
               w1x, s1f, t1f, w2f, b2f, s2f, t2f,
               wg1, bg1, sg1, tg1, wg2, bg2, sg2, tg2, w1p,
               out_ref, *, vblk):
    eblk = vblk * _K
    rep = jnp.broadcast_to(dt_ref[...][:, None, :], (vblk, _K, _CD))
    rep = rep.reshape(eblk, _CD)
    z1 = _mm(xs_ref[...], w1x[...]) + _mm(ps_ref[...], w1p[...]) + rep
    y1 = jnp.maximum(z1, 0.0) * s1f[...] + t1f[...]
    z2 = _mm(y1, w2f[...]) + b2f[...]
    msg = jnp.maximum(z2, 0.0) * s2f[...] + t2f[...]
    agg = jnp.max(msg.reshape(vblk, _K, _CD), axis=1)
    z3 = _mm(agg, wg1[...]) + bg1[...]
    y3 = jnp.maximum(z3, 0.0) * sg1[...] + tg1[...]
    z4 = _mm(y3, wg2[...]) + bg2[...]
    y4 = jnp.maximum(z4, 0.0) * sg2[...] + tg2[...]
    out_ref[...] = h_ref[...] + y4


def _mha_body(inp_ref, x_ref, hs_ref, sn, tn, wqT, bq, bdk, bk, bdv, bv,
              t16, g8, e8, hmat, woT, bo, out_ref):
    hs = hs_ref[...] * sn[...] + tn[...]
    q = _mm(x_ref[...], wqT[...]) + bq[...]
    kf = _mm(hs, bdk[...]) + bk[...]
    vf = _mm(hs, bdv[...]) + bv[...]
    qrep = _mm(q, t16[...])
    lg = _mm(qrep * kf, g8[...]) * 0.25
    mx = jnp.max(lg, axis=1, keepdims=True)
    e = jnp.exp(lg - mx)
    a = e / jnp.sum(e, axis=1, keepdims=True)
    o16 = _mm(_mm(a, e8[...]) * vf, hmat[...])
    o = _mm(o16, woT[...]) + bo[...]
    out_ref[...] = jnp.concatenate([inp_ref[...], o], axis=1)


def _bn_fold(p, eps):
    s = p["gamma"] / jnp.sqrt(p["var"] + eps)
    t = p["beta"] - p["mean"] * s
    return s, t


def _row(v):
    return v.reshape(1, -1)


def _mymlp_fold(layers):
    out = []
    for l in layers:
        s, t = _bn_fold(l["bn"], 1e-5)
        out.append((l["lin"]["W"].T, _row(l["lin"]["b"]), _row(s), _row(t)))
    return out


def _full_spec(shape):
    nd = len(shape)
    return pl.BlockSpec(shape, lambda i, _n=nd: (0,) * _n)


def _blk_spec(shape):
    return pl.BlockSpec(shape, lambda i: (i, 0))


def _call(body, grid, in_arrays, in_specs, out_shapes, out_specs, **kw):
    return pl.pallas_call(
        functools.partial(body, **kw) if kw else body,
        grid=grid,
        in_specs=in_specs,
        out_specs=out_specs,
        out_shape=out_shapes,
    )(*in_arrays)


def kernel(inp, inverse, coords, bev_shape, params):
    f32 = jnp.float32
    eye8 = jnp.eye(_KL, dtype=f32)
    eye16 = jnp.eye(_DIM, dtype=f32)
    e8 = jnp.kron(eye8, jnp.ones((1, _DIM), f32))
    e8x = jnp.concatenate([e8, jnp.zeros((8, _CD), f32)], axis=0)
    hmat = jnp.kron(jnp.ones((_KL, 1), f32), eye16)
    t16 = hmat.T
    g8 = e8.T
    sel816 = jnp.concatenate([eye8, jnp.zeros((_KL, _KL), f32)], axis=1)

    pre = []
    for l in params["pre_mlp"]:
        s, t = _bn_fold(l["bn"], 1e-3)
        pre.append((l["lin"]["W"].T * s[None, :], _row(l["lin"]["b"] * s + t)))
    wsT = params["score"]["W"].T
    bs = _row(params["score"]["b"])

    blk_a = 1792
    ngrid_a = _NP // blk_a
    widen = jnp.eye(_DIM, _CD, dtype=f32)
    consts_a = (pre[0][0], pre[0][1], pre[1][0], pre[1][1], pre[2][0],
                pre[2][1], wsT, bs, sel816, e8x, t16, widen)
    x, u_vals, ex_vals = _call(
        _premlp_body, (ngrid_a,),
        [inp, *consts_a],
        [_blk_spec((blk_a, _DIM))] + [_full_spec(a.shape) for a in consts_a],
        (jax.ShapeDtypeStruct((_NP, _DIM), f32),
         jax.ShapeDtypeStruct((_NP, _CD), f32),
         jax.ShapeDtypeStruct((_NP, _CD), f32)),
        (_blk_spec((blk_a, _DIM)), _blk_spec((blk_a, _CD)),
         _blk_spec((blk_a, _CD))),
        blk=blk_a,
    )

    pad_n = _NP - _N
    inv_pad = jnp.concatenate(
        [inverse.astype(jnp.int32), jnp.zeros((pad_n,), jnp.int32)])
    inv3s = inv_pad.reshape(_NS, _SC_N * _SC_CH)
    inv3 = inv_pad.reshape(_NW, _NP_NCH * _NP_CH)
    z128 = jnp.zeros((_VROWS, _CD), f32)
    p1, p2 = _sc_scatter_add(u_vals, ex_vals, inv3s, z128)

    selex = jnp.concatenate([e8x, jnp.zeros((_CD - _DIM, _CD), f32)], axis=0)
    h = _call(
        _combine_body, (1,),
        [p1, p2, selex],
        [_full_spec(p1.shape), _full_spec(p2.shape), _full_spec(selex.shape)],
        jax.ShapeDtypeStruct((_VP, _CD), f32),
        _full_spec((_VP, _CD)),
    )

    pos = coords[:, 1:4]
    pos16 = jnp.zeros((_VP, _DIM), f32).at[:_V, :3].set(pos)
    posT = pos16.T
    kblk = 1000
    knn = _call(
        _knn_body, (_V // kblk,),
        [pos16[:_V], posT],
        [_blk_spec((kblk, _DIM)), _full_spec(posT.shape)],
        jax.ShapeDtypeStruct((_V, 8), jnp.int32),
        _blk_spec((kblk, 8)),
        blk=kblk,
    )
    src = knn[:, :_K].reshape(-1)
    src3 = jnp.concatenate(
        [src, jnp.zeros((_EP - src.shape[0],), jnp.int32)]
    ).reshape(_NW, _EP_NCH * _EP_CH)

    pos128 = jnp.zeros((_VP, _CD), f32).at[:_V, :3].set(pos)
    pos_src = _sc_gather(pos128, src3, nch=_EP_NCH, ch=_EP_CH)

    vblk = 256
    ngrid = _VP // vblk
    for lp in params["gnn"]:
        hf = _mymlp_fold(lp["h"])
        ff = _mymlp_fold(lp["f"])
        gf = _mymlp_fold(lp["g"])
        (wh1, bh1, s1, t1), (wh2r, bh2r, s2r, t2r) = hf
        wh2 = jnp.zeros((64, _DIM), f32).at[:, :3].set(wh2r)
        bh2 = jnp.zeros((1, _DIM), f32).at[:, :3].set(bh2r)
        s2 = jnp.zeros((1, _DIM), f32).at[:, :3].set(s2r)
        t2 = jnp.zeros((1, _DIM), f32).at[:, :3].set(t2r)
        (w1T, b1, s1f, t1f), (w2f, b2f, s2f, t2f) = ff
        w1p = jnp.zeros((_DIM, _CD), f32).at[:3, :].set(w1T[:3, :])
        w1pe = jnp.zeros((_CD, _CD), f32).at[:3, :].set(w1T[:3, :])
        w1x = w1T[3:, :]
        (wg1, bg1, sg1, tg1), (wg2, bg2, sg2, tg2) = gf

        consts_d = (wh1, bh1, s1, t1, wh2, bh2, s2, t2, w1p, b1)
        dstterm = _call(
            _delta_body, (1,),
            [h, pos16, *consts_d],
            [_full_spec((_VP, _CD)), _full_spec((_VP, _DIM))] +
            [_full_spec(a.shape) for a in consts_d],
            jax.ShapeDtypeStruct((_VP, _CD), f32),
            _full_spec((_VP, _CD)),
        )
        x_src = _sc_gather(h, src3, nch=_EP_NCH, ch=_EP_CH)
        consts_e = (w1x, s1f, t1f, w2f, b2f, s2f, t2f,
                    wg1, bg1, sg1, tg1, wg2, bg2, sg2, tg2, w1pe)
        h = _call(
            _edge_body, (ngrid,),
            [h, dstterm, x_src, pos_src, *consts_e],
            [_blk_spec((vblk, _CD)), _blk_spec((vblk, _CD)),
             _blk_spec((vblk * _K, _CD)), _blk_spec((vblk * _K, _CD))] +
            [_full_spec(a.shape) for a in consts_e],
            jax.ShapeDtypeStruct((_VP, _CD), f32),
            _blk_spec((vblk, _CD)),
            vblk=vblk,
        )

    hs = _sc_gather(h, inv3, nch=_NP_NCH, ch=_NP_CH)

    sn, tn = _bn_fold(params["norm"], 1e-3)
    sn128 = _row(jnp.tile(sn, _KL))
    tn128 = _row(jnp.tile(tn, _KL))
    m = params["mha"]
    wq, wk, wv = jnp.split(m["in_W"], 3, axis=0)
    bq, bk, bv = jnp.split(m["in_b"], 3)
    bdk = jnp.kron(eye8, wk.T)
    bdv = jnp.kron(eye8, wv.T)
    bk128 = _row(jnp.tile(bk, _KL))
    bv128 = _row(jnp.tile(bv, _KL))
    consts_g = (sn128, tn128, wq.T, _row(bq), bdk, bk128, bdv, bv128,
                t16, g8, e8, hmat, m["out_W"].T, _row(m["out_b"]))
    nblk, blk = 25, 2000
    out = _call(
        _mha_body, (nblk,),
        [inp, x, hs, *consts_g],
        [_blk_spec((blk, _DIM)), _blk_spec((blk, _DIM)),
         _blk_spec((blk, _CD))] +
        [_full_spec(a.shape) for a in consts_g],
        jax.ShapeDtypeStruct((_N, 2 * _DIM), f32),
        _blk_spec((blk, 2 * _DIM)),
    )
    return out


# speedup vs baseline: 7.7875x; 1.0209x over previous
import functools

import jax
import jax.numpy as jnp
from jax import lax
from jax.experimental import pallas as pl
from jax.experimental.pallas import tpu as pltpu
from jax.experimental.pallas import tpu_sc as plsc

_DIM = 16
_KL = 8
_CD = 128
_N = 50000
_V = 5000
_K = 6

_NC = 2
_NS = 16
_NW = _NC * _NS

_SC_CH = 224
_SC_N = 14
_NP_CH = 224
_NP_NCH = 7
_NP = _NW * _NP_NCH * _NP_CH
_VP = 5120
_VROWS = _VP // _NS
_EP_CH = 192
_EP_NCH = 5
_EP = _NW * _EP_NCH * _EP_CH

@functools.cache
def _mesh():
    return plsc.VectorSubcoreMesh(
        core_axis_name="c", subcore_axis_name="s",
        num_cores=_NC, num_subcores=_NS,
    )


def _sc_gather(table, idx2, *, nch, ch):
    d = table.shape[1]
    per_tile = nch * ch

    @functools.partial(
        pl.kernel,
        out_type=jax.ShapeDtypeStruct((_NW * per_tile, d), jnp.float32),
        mesh=_mesh(),
        scratch_types=[
            pltpu.VMEM((per_tile,), jnp.int32),
            pltpu.VMEM((ch, d), jnp.float32),
            pltpu.VMEM((ch, d), jnp.float32),
            pltpu.VMEM_SHARED((_VP, d), jnp.float32),
            pltpu.SemaphoreType.DMA,
            pltpu.SemaphoreType.DMA,
        ],
    )
    def body(table_hbm, idx_hbm, out_hbm, idx_full, rows0, rows1, tab_sh,
             sem0, sem1):
        cid = lax.axis_index("c")
        sid = lax.axis_index("s")
        wid = sid * _NC + cid
        base = wid * per_tile
        pltpu.sync_copy(table_hbm.at[pl.ds(sid * _VROWS, _VROWS)],
                        tab_sh.at[pl.ds(sid * _VROWS, _VROWS)])
        pltpu.sync_copy(idx_hbm.at[wid], idx_full)
        plsc.subcore_barrier()
        rows = (rows0, rows1)
        sems = (sem0, sem1)
        cps = [None] * nch
        cps[0] = pltpu.async_copy(
            tab_sh.at[idx_full.at[pl.ds(0, ch)]], rows0, sem0)
        for c in range(nch):
            b = c & 1
            nb = 1 - b
            if c + 1 < nch:
                cps[c + 1] = pltpu.async_copy(
                    tab_sh.at[idx_full.at[pl.ds((c + 1) * ch, ch)]],
                    rows[nb], sems[nb])
            cps[c].wait()
            pltpu.sync_copy(rows[b], out_hbm.at[pl.ds(base + c * ch, ch)])

    return body(table, idx2)


def _sc_scatter_add(u_vals, ex_vals, idx2, zeros128):
    @functools.partial(
        pl.kernel,
        out_type=(
            jax.ShapeDtypeStruct((_VP, _CD), jnp.float32),
            jax.ShapeDtypeStruct((_VP, _CD), jnp.float32),
        ),
        mesh=_mesh(),
        scratch_types=[
            pltpu.VMEM((_SC_N * _SC_CH,), jnp.int32),
            pltpu.VMEM((_SC_CH, _CD), jnp.float32),
            pltpu.VMEM((_SC_CH, _CD), jnp.float32),
            pltpu.VMEM_SHARED((_VP, _CD), jnp.float32),
            pltpu.SemaphoreType.DMA,
            pltpu.SemaphoreType.DMA,
        ],
    )
    def body(u_hbm, ex_hbm, idx_hbm, z_hbm, out1_hbm, out2_hbm,
             idx_full, v0, v1, acc_sh, sem0, sem1):
        cid = lax.axis_index("c")
        sid = lax.axis_index("s")
        row0 = sid * _VROWS
        pltpu.sync_copy(z_hbm, acc_sh.at[pl.ds(row0, _VROWS)])
        plsc.subcore_barrier()
        pltpu.sync_copy(idx_hbm.at[sid], idx_full)
        base = sid * (_SC_N * _SC_CH)
        bufs = (v0, v1)
        sems = (sem0, sem1)

        def run(vals_hbm, out_hbm):
            cps = [None] * _SC_N
            cps[0] = pltpu.async_copy(
                vals_hbm.at[pl.ds(base, _SC_CH)], v0, sem0)
            for c in range(_SC_N):
                b = c & 1
                nb = 1 - b
                if c + 1 < _SC_N:
                    cps[c + 1] = pltpu.async_copy(
                        vals_hbm.at[pl.ds(base + (c + 1) * _SC_CH, _SC_CH)],
                        bufs[nb], sems[nb])
                cps[c].wait()
                pltpu.sync_copy(
                    bufs[b],
                    acc_sh.at[idx_full.at[pl.ds(c * _SC_CH, _SC_CH)]],
                    add=True)
            plsc.subcore_barrier()
            pltpu.sync_copy(acc_sh.at[pl.ds(row0, _VROWS)],
                            out_hbm.at[pl.ds(row0, _VROWS)])

        @pl.when(cid == 0)
        def _():
            run(u_hbm, out1_hbm)

        @pl.when(cid == 1)
        def _():
            run(ex_hbm, out2_hbm)

    return body(u_vals, ex_vals, idx2, zeros128)


def _mm(a, b):
    return jnp.dot(a, b, preferred_element_type=jnp.float32)


def _premlp_body(inp_ref, w0, b0, w1, b1, w2, b2, wsT, bs, sel816, e8x, t16,
                 widen, x_ref, u_ref, ex_ref, *, blk):
    pid = pl.program_id(0)
    x = inp_ref[...]
    x = jnp.maximum(_mm(x, w0[...]) + b0[...], 0.0)
    x = jnp.maximum(_mm(x, w1[...]) + b1[...], 0.0)
    x = _mm(x, w2[...]) + b2[...]
    score = _mm(x, wsT[...]) + bs[...]
    valid = (lax.broadcasted_iota(jnp.int32, (blk, 1), 0) + pid * blk) < _N
    ex = jnp.where(valid, jnp.exp(score), 0.0)
    ex16 = _mm(ex, sel816[...])
    u = _mm(ex16, e8x[...]) * _mm(x, t16[...])
    x_ref[...] = x
    u_ref[...] = jnp.where(valid, u, 0.0)
    ex_ref[...] = _mm(ex16, widen[...])


def _combine_body(p1_ref, p2_ref, selex_ref, h_ref):
    u = p1_ref[...]
    s2 = p2_ref[...]
    denom = _mm(s2, selex_ref[...])
    denom = jnp.where(denom == 0.0, 1.0, denom)
    h_ref[...] = jnp.maximum(u / denom, 0.0)


def _knn_body(pos_ref, posT_ref, out_ref, *, blk):
    i = pl.program_id(0)
    pb = pos_ref[...]
    pT = posT_ref[...]
    sqb = jnp.sum(pb * pb, axis=1, keepdims=True)
    sqa = jnp.sum(pT * pT, axis=0, keepdims=True)
    d = sqb + sqa - 2.0 * _mm(pb, pT)
    col = lax.broadcasted_iota(jnp.int32, (blk, _VP), 1)
    row = lax.broadcasted_iota(jnp.int32, (blk, _VP), 0) + i * blk
    d = jnp.where(col == row, d + 1e10, d)
    d = jnp.where(col >= _V, 3e38, d)
    res = jnp.zeros((blk, 8), jnp.int32)
    lane8 = lax.broadcasted_iota(jnp.int32, (blk, 8), 1)
    for t in range(_K):
        am = jnp.argmin(d, axis=1).astype(jnp.int32)
        res = jnp.where(lane8 == t, am[:, None], res)
        d = jnp.where(col == am[:, None], 3e38, d)
    out_ref[...] = res


def _delta_body(h_ref, pos_ref, wh1, bh1, s1, t1, wh2, bh2, s2, t2,
                w1p, b1f, w1x, out_ref, a_ref):
    z1 = _mm(h_ref[...], wh1[...]) + bh1[...]
    y1 = jnp.maximum(z1, 0.0) * s1[...] + t1[...]
    z2 = _mm(y1, wh2[...]) + bh2[...]
    delta = jnp.maximum(z2, 0.0) * s2[...] + t2[...]
    out_ref[...] = _mm(delta - pos_ref[...], w1p[...]) + b1f[...]
    a_ref[...] = _mm(h_ref[...], w1x[...]) + _mm(pos_ref[...], w1p[...])


def _edge_body(h_ref, dt_ref, xs_ref,
               s1f, t1f, w2f, b2f, s2f, t2f,
               wg1, bg1, sg1, tg1, wg2, bg2, sg2, tg2,
               out_ref, *, vblk):
    eblk = vblk * _K
    rep = jnp.broadcast_to(dt_ref[...][:, None, :], (vblk, _K, _CD))
    rep = rep.reshape(eblk, _CD)
    z1 = xs_ref[...] + rep
    y1 = jnp.maximum(z1, 0.0) * s1f[...] + t1f[...]
    z2 = _mm(y1, w2f[...]) + b2f[...]
    msg = jnp.maximum(z2, 0.0) * s2f[...] + t2f[...]
    agg = jnp.max(msg.reshape(vblk, _K, _CD), axis=1)
    z3 = _mm(agg, wg1[...]) + bg1[...]
    y3 = jnp.maximum(z3, 0.0) * sg1[...] + tg1[...]
    z4 = _mm(y3, wg2[...]) + bg2[...]
    y4 = jnp.maximum(z4, 0.0) * sg2[...] + tg2[...]
    out_ref[...] = h_ref[...] + y4


def _mha_body(inp_ref, x_ref, hs_ref, sn, tn, wqT, bq, bdk, bk, bdv, bv,
              t16, g8, e8, hmat, woT, bo, out_ref):
    hs = hs_ref[...] * sn[...] + tn[...]
    q = _mm(x_ref[...], wqT[...]) + bq[...]
    kf = _mm(hs, bdk[...]) + bk[...]
    vf = _mm(hs, bdv[...]) + bv[...]
    qrep = _mm(q, t16[...])
    lg = _mm(qrep * kf, g8[...]) * 0.25
    mx = jnp.max(lg, axis=1, keepdims=True)
    e = jnp.exp(lg - mx)
    a = e / jnp.sum(e, axis=1, keepdims=True)
    o16 = _mm(_mm(a, e8[...]) * vf, hmat[...])
    o = _mm(o16, woT[...]) + bo[...]
    out_ref[...] = jnp.concatenate([inp_ref[...], o], axis=1)


def _bn_fold(p, eps):
    s = p["gamma"] / jnp.sqrt(p["var"] + eps)
    t = p["beta"] - p["mean"] * s
    return s, t


def _row(v):
    return v.reshape(1, -1)


def _mymlp_fold(layers):
    out = []
    for l in layers:
        s, t = _bn_fold(l["bn"], 1e-5)
        out.append((l["lin"]["W"].T, _row(l["lin"]["b"]), _row(s), _row(t)))
    return out


def _full_spec(shape):
    nd = len(shape)
    return pl.BlockSpec(shape, lambda i, _n=nd: (0,) * _n)


def _blk_spec(shape):
    return pl.BlockSpec(shape, lambda i: (i, 0))


def _call(body, grid, in_arrays, in_specs, out_shapes, out_specs, **kw):
    return pl.pallas_call(
        functools.partial(body, **kw) if kw else body,
        grid=grid,
        in_specs=in_specs,
        out_specs=out_specs,
        out_shape=out_shapes,
    )(*in_arrays)


def kernel(inp, inverse, coords, bev_shape, params):
    f32 = jnp.float32
    eye8 = jnp.eye(_KL, dtype=f32)
    eye16 = jnp.eye(_DIM, dtype=f32)
    e8 = jnp.kron(eye8, jnp.ones((1, _DIM), f32))
    e8x = jnp.concatenate([e8, jnp.zeros((8, _CD), f32)], axis=0)
    hmat = jnp.kron(jnp.ones((_KL, 1), f32), eye16)
    t16 = hmat.T
    g8 = e8.T
    sel816 = jnp.concatenate([eye8, jnp.zeros((_KL, _KL), f32)], axis=1)

    pre = []
    for l in params["pre_mlp"]:
        s, t = _bn_fold(l["bn"], 1e-3)
        pre.append((l["lin"]["W"].T * s[None, :], _row(l["lin"]["b"] * s + t)))
    wsT = params["score"]["W"].T
    bs = _row(params["score"]["b"])

    blk_a = 1792
    ngrid_a = _NP // blk_a
    widen = jnp.eye(_DIM, _CD, dtype=f32)
    consts_a = (pre[0][0], pre[0][1], pre[1][0], pre[1][1], pre[2][0],
                pre[2][1], wsT, bs, sel816, e8x, t16, widen)
    x, u_vals, ex_vals = _call(
        _premlp_body, (ngrid_a,),
        [inp, *consts_a],
        [_blk_spec((blk_a, _DIM))] + [_full_spec(a.shape) for a in consts_a],
        (jax.ShapeDtypeStruct((_NP, _DIM), f32),
         jax.ShapeDtypeStruct((_NP, _CD), f32),
         jax.ShapeDtypeStruct((_NP, _CD), f32)),
        (_blk_spec((blk_a, _DIM)), _blk_spec((blk_a, _CD)),
         _blk_spec((blk_a, _CD))),
        blk=blk_a,
    )

    pad_n = _NP - _N
    inv_pad = jnp.concatenate(
        [inverse.astype(jnp.int32), jnp.zeros((pad_n,), jnp.int32)])
    inv3s = inv_pad.reshape(_NS, _SC_N * _SC_CH)
    inv3 = inv_pad.reshape(_NW, _NP_NCH * _NP_CH)
    z128 = jnp.zeros((_VROWS, _CD), f32)
    p1, p2 = _sc_scatter_add(u_vals, ex_vals, inv3s, z128)

    selex = jnp.concatenate([e8x, jnp.zeros((_CD - _DIM, _CD), f32)], axis=0)
    h = _call(
        _combine_body, (1,),
        [p1, p2, selex],
        [_full_spec(p1.shape), _full_spec(p2.shape), _full_spec(selex.shape)],
        jax.ShapeDtypeStruct((_VP, _CD), f32),
        _full_spec((_VP, _CD)),
    )

    pos = coords[:, 1:4]
    pos16 = jnp.zeros((_VP, _DIM), f32).at[:_V, :3].set(pos)
    posT = pos16.T
    kblk = 1000
    knn = _call(
        _knn_body, (_V // kblk,),
        [pos16[:_V], posT],
        [_blk_spec((kblk, _DIM)), _full_spec(posT.shape)],
        jax.ShapeDtypeStruct((_V, 8), jnp.int32),
        _blk_spec((kblk, 8)),
        blk=kblk,
    )
    src = knn[:, :_K].reshape(-1)
    src3 = jnp.concatenate(
        [src, jnp.zeros((_EP - src.shape[0],), jnp.int32)]
    ).reshape(_NW, _EP_NCH * _EP_CH)

    vblk = 256
    ngrid = _VP // vblk
    for lp in params["gnn"]:
        hf = _mymlp_fold(lp["h"])
        ff = _mymlp_fold(lp["f"])
        gf = _mymlp_fold(lp["g"])
        (wh1, bh1, s1, t1), (wh2r, bh2r, s2r, t2r) = hf
        wh2 = jnp.zeros((64, _DIM), f32).at[:, :3].set(wh2r)
        bh2 = jnp.zeros((1, _DIM), f32).at[:, :3].set(bh2r)
        s2 = jnp.zeros((1, _DIM), f32).at[:, :3].set(s2r)
        t2 = jnp.zeros((1, _DIM), f32).at[:, :3].set(t2r)
        (w1T, b1, s1f, t1f), (w2f, b2f, s2f, t2f) = ff
        w1p = jnp.zeros((_DIM, _CD), f32).at[:3, :].set(w1T[:3, :])
        w1x = w1T[3:, :]
        (wg1, bg1, sg1, tg1), (wg2, bg2, sg2, tg2) = gf

        consts_d = (wh1, bh1, s1, t1, wh2, bh2, s2, t2, w1p, b1, w1x)
        dstterm, a_tab = _call(
            _delta_body, (1,),
            [h, pos16, *consts_d],
            [_full_spec((_VP, _CD)), _full_spec((_VP, _DIM))] +
            [_full_spec(a.shape) for a in consts_d],
            (jax.ShapeDtypeStruct((_VP, _CD), f32),
             jax.ShapeDtypeStruct((_VP, _CD), f32)),
            (_full_spec((_VP, _CD)), _full_spec((_VP, _CD))),
        )
        x_src = _sc_gather(a_tab, src3, nch=_EP_NCH, ch=_EP_CH)
        consts_e = (s1f, t1f, w2f, b2f, s2f, t2f,
                    wg1, bg1, sg1, tg1, wg2, bg2, sg2, tg2)
        h = _call(
            _edge_body, (ngrid,),
            [h, dstterm, x_src, *consts_e],
            [_blk_spec((vblk, _CD)), _blk_spec((vblk, _CD)),
             _blk_spec((vblk * _K, _CD))] +
            [_full_spec(a.shape) for a in consts_e],
            jax.ShapeDtypeStruct((_VP, _CD), f32),
            _blk_spec((vblk, _CD)),
            vblk=vblk,
        )

    hs = _sc_gather(h, inv3, nch=_NP_NCH, ch=_NP_CH)

    sn, tn = _bn_fold(params["norm"], 1e-3)
    sn128 = _row(jnp.tile(sn, _KL))
    tn128 = _row(jnp.tile(tn, _KL))
    m = params["mha"]
    wq, wk, wv = jnp.split(m["in_W"], 3, axis=0)
    bq, bk, bv = jnp.split(m["in_b"], 3)
    bdk = jnp.kron(eye8, wk.T)
    bdv = jnp.kron(eye8, wv.T)
    bk128 = _row(jnp.tile(bk, _KL))
    bv128 = _row(jnp.tile(bv, _KL))
    consts_g = (sn128, tn128, wq.T, _row(bq), bdk, bk128, bdv, bv128,
                t16, g8, e8, hmat, m["out_W"].T, _row(m["out_b"]))
    nblk, blk = 25, 2000
    out = _call(
        _mha_body, (nblk,),
        [inp, x, hs, *consts_g],
        [_blk_spec((blk, _DIM)), _blk_spec((blk, _DIM)),
         _blk_spec((blk, _CD))] +
        [_full_spec(a.shape) for a in consts_g],
        jax.ShapeDtypeStruct((_N, 2 * _DIM), f32),
        _blk_spec((blk, 2 * _DIM)),
    )
    return out
